# baseline (speedup 1.0000x reference)
import functools

import jax
import jax.numpy as jnp
from jax import lax
from jax.experimental import pallas as pl
from jax.experimental.pallas import tpu as pltpu
from jax.experimental.pallas import tpu_sc as plsc

B, N, D, NK = 4, 4096, 64, 32
OC = 32
TW = 48
R = B * N
M = B * N * NK
NC, NS, L = 2, 16, 16
NW = NC * NS
ROWS_PER_W = R // NW
FMAX = 3.4e38


def _k1_body(xf_ref, fl_ref, wa_ref, wc_ref, b1_ref, t_ref, c_ref):
    xf = xf_ref[0]
    a = lax.dot_general(xf, wa_ref[...], (((0,), (0,)), ((), ())),
                        preferred_element_type=jnp.float32)
    c = lax.dot_general(xf, wc_ref[...], (((0,), (0,)), ((), ())),
                        preferred_element_type=jnp.float32) + b1_ref[...]
    t_ref[0, :, 0:OC] = a
    t_ref[0, :, OC:OC + 3] = fl_ref[0].T
    t_ref[0, :, OC + 3:TW] = jnp.zeros((a.shape[0], TW - OC - 3), jnp.float32)
    c_ref[0] = c


def _k1(xf, flow, wa, wc, b1):
    TN = 1024
    grid = (B, N // TN)
    return pl.pallas_call(
        _k1_body,
        grid=grid,
        in_specs=[
            pl.BlockSpec((1, 3 + D, TN), lambda b, i: (b, 0, i)),
            pl.BlockSpec((1, 3, TN), lambda b, i: (b, 0, i)),
            pl.BlockSpec((3 + D, OC), lambda b, i: (0, 0)),
            pl.BlockSpec((3 + D, OC), lambda b, i: (0, 0)),
            pl.BlockSpec((1, OC), lambda b, i: (0, 0)),
        ],
        out_specs=[
            pl.BlockSpec((1, TN, TW), lambda b, i: (b, i, 0)),
            pl.BlockSpec((1, TN, OC), lambda b, i: (b, i, 0)),
        ],
        out_shape=[
            jax.ShapeDtypeStruct((B, N, TW), jnp.float32),
            jax.ShapeDtypeStruct((B, N, OC), jnp.float32),
        ],
    )(xf, flow, wa, wc, b1[None])


def _k1_wrap(pc1, feat1, flow, W1, b1):
    w1x = W1[:, 0:3]
    w1f = W1[:, 3:3 + D]
    w1g = W1[:, 3 + D:]
    wa = jnp.concatenate([w1x, w1f], axis=1).T
    wc = jnp.concatenate([-w1x, w1g], axis=1).T
    xf = jnp.concatenate([pc1, feat1], axis=1)
    t, c = _k1(xf, flow, wa, wc, b1)
    return t.reshape(R, TW), c.reshape(R, OC)


def _bf16_round(x):
    u = plsc.bitcast(x, jnp.uint32)
    r = u + jnp.uint32(0x7FFF) + ((u >> jnp.uint32(16)) & jnp.uint32(1))
    return plsc.bitcast(r & jnp.uint32(0xFFFF0000), jnp.float32)


def _k2_body(pc1_hbm, t_hbm, out_hbm, px, py, pz,
             pxb, pyb, pzb, s2, idxbuf, rowbuf, sem):
    wid = lax.axis_index("s") * NC + lax.axis_index("c")
    wpb = NW // B
    b = wid // wpb
    pltpu.sync_copy(pc1_hbm.at[pl.ds(b * 3 * N, N)], px)
    pltpu.sync_copy(pc1_hbm.at[pl.ds(b * 3 * N + N, N)], py)
    pltpu.sync_copy(pc1_hbm.at[pl.ds(b * 3 * N + 2 * N, N)], pz)
    row0 = wid * ROWS_PER_W
    n0 = (wid % wpb) * ROWS_PER_W
    gbase = b * N

    def pre_body(ci, _):
        base = ci * L
        cx = px[pl.ds(base, L)]
        cy = py[pl.ds(base, L)]
        cz = pz[pl.ds(base, L)]
        pxb[pl.ds(base, L)] = _bf16_round(cx)
        pyb[pl.ds(base, L)] = _bf16_round(cy)
        pzb[pl.ds(base, L)] = _bf16_round(cz)
        s2[pl.ds(base, L)] = cx * cx + cy * cy + cz * cz
        return 0

    lax.fori_loop(0, N // L, pre_body, 0, unroll=False)

    def row_body(r, _):
        n = n0 + r
        nsplat = jnp.zeros((L,), jnp.int32) + n
        pxn = plsc.load_gather(pxb, [nsplat])
        pyn = plsc.load_gather(pyb, [nsplat])
        pzn = plsc.load_gather(pzb, [nsplat])
        sn2 = plsc.load_gather(s2, [nsplat])

        def chunk_body(ci, carry):
            base = ci * L
            cx = pxb[pl.ds(base, L)]
            cy = pyb[pl.ds(base, L)]
            cz = pzb[pl.ds(base, L)]
            dot = cx * pxn + cy * pyn + cz * pzn
            d = (jnp.float32(-2.0) * dot + sn2) + s2[pl.ds(base, L)]
            r0v, r0i, r1v, r1i = carry
            hit = jnp.any(d < r1v[L - 1])

            def merge(c):
                r0v, r0i, r1v, r1i = c
                rcv, rci = plsc.sort_key_val(
                    d, lax.iota(jnp.int32, L) + base, descending=True)
                m = r1v <= rcv
                lv = jnp.where(m, r1v, rcv)
                li = jnp.where(m, r1i, rci)
                rlv, rli = plsc.sort_key_val(lv, li, descending=True)
                m2 = r0v <= rlv
                av = jnp.where(m2, r0v, rlv)
                ai = jnp.where(m2, r0i, rli)
                bv = jnp.where(m2, rlv, r0v)
                bi = jnp.where(m2, rli, r0i)
                nr0 = plsc.sort_key_val(av, ai)
                nr1 = plsc.sort_key_val(bv, bi)
                return nr0[0], nr0[1], nr1[0], nr1[1]

            return lax.cond(hit, merge, lambda c: c, carry)

        init = (jnp.full((L,), FMAX, jnp.float32), jnp.zeros((L,), jnp.int32),
                jnp.full((L,), FMAX, jnp.float32), jnp.zeros((L,), jnp.int32))
        r0v, r0i, r1v, r1i = lax.fori_loop(0, N // L, chunk_body, init,
                                           unroll=False)
        c0 = pltpu.async_copy(t_hbm.at[r0i + gbase],
                              rowbuf.at[pl.ds(0, L)], sem)
        c1 = pltpu.async_copy(t_hbm.at[r1i + gbase],
                              rowbuf.at[pl.ds(L, L)], sem)
        c0.wait()
        c1.wait()
        pltpu.sync_copy(rowbuf, out_hbm.at[row0 + r])
        return 0

    lax.fori_loop(0, ROWS_PER_W, row_body, 0, unroll=False)


def _knn_gather_sc(pc1, t_tab):
    mesh = plsc.VectorSubcoreMesh(core_axis_name="c", subcore_axis_name="s")
    f = functools.partial(
        pl.kernel,
        out_type=jax.ShapeDtypeStruct((R, NK, TW), jnp.float32),
        mesh=mesh,
        scratch_types=[
            pltpu.VMEM((N,), jnp.float32),
            pltpu.VMEM((N,), jnp.float32),
            pltpu.VMEM((N,), jnp.float32),
            pltpu.VMEM((N,), jnp.float32),
            pltpu.VMEM((N,), jnp.float32),
            pltpu.VMEM((N,), jnp.float32),
            pltpu.VMEM((N,), jnp.float32),
            pltpu.VMEM((NK,), jnp.int32),
            pltpu.VMEM((NK, TW), jnp.float32),
            pltpu.SemaphoreType.DMA,
        ],
        compiler_params=pltpu.CompilerParams(needs_layout_passes=False,
                                             use_tc_tiling_on_sc=False),
    )(_k2_body)
    return f(pc1.reshape(B * 3 * N), t_tab)


def _k3a_body(g_ref, c_ref, acc_ref):
    step = pl.program_id(0)

    @pl.when(step == 0)
    def _():
        acc_ref[...] = jnp.zeros_like(acc_ref)

    h = g_ref[:, :, 0:OC] + c_ref[...][:, None, :]
    acc_ref[0:1, :] += jnp.sum(h, axis=(0, 1)).reshape(1, OC)
    acc_ref[1:2, :] += jnp.sum(h * h, axis=(0, 1)).reshape(1, OC)


def _k3b_body(g_ref, c_ref, fl_ref, w2_ref, gb_ref, acc_ref, o_ref):
    h = g_ref[:, :, 0:OC] + c_ref[...][:, None, :]
    mean = acc_ref[0:1, :].reshape(1, 1, OC) * (1.0 / M)
    ex2 = acc_ref[1:2, :].reshape(1, 1, OC) * (1.0 / M)
    var = ex2 - mean * mean
    gamma = w2_ref[1:2, :].reshape(1, 1, OC)
    beta = w2_ref[2:3, :].reshape(1, 1, OC)
    w2 = w2_ref[0:1, :].reshape(1, 1, OC)
    scale = gamma * lax.rsqrt(var + 1e-5)
    hn = (h - mean) * scale + beta
    hn = jnp.where(hn >= 0, hn, 0.1 * hn)
    s = jnp.sum(hn * w2, axis=2) + gb_ref[0, 0]
    s = s - jnp.max(s, axis=1, keepdims=True)
    e = jnp.exp(s)
    w = e / jnp.sum(e, axis=1, keepdims=True)
    gfl = g_ref[:, :, OC:OC + 3]
    o_ref[...] = fl_ref[...] + jnp.sum(w[:, :, None] * gfl, axis=1)


def _k3(g, c, fl_t, w2row, b2):
    TN = 512
    nb = R // TN
    acc = pl.pallas_call(
        _k3a_body,
        grid=(nb,),
        in_specs=[
            pl.BlockSpec((TN, NK, TW), lambda i: (i, 0, 0)),
            pl.BlockSpec((TN, OC), lambda i: (i, 0)),
        ],
        out_specs=pl.BlockSpec((2, OC), lambda i: (0, 0)),
        out_shape=jax.ShapeDtypeStruct((2, OC), jnp.float32),
    )(g, c)
    return pl.pallas_call(
        _k3b_body,
        grid=(nb,),
        in_specs=[
            pl.BlockSpec((TN, NK, TW), lambda i: (i, 0, 0)),
            pl.BlockSpec((TN, OC), lambda i: (i, 0)),
            pl.BlockSpec((TN, 3), lambda i: (i, 0)),
            pl.BlockSpec((3, OC), lambda i: (0, 0)),
            pl.BlockSpec((1, 1), lambda i: (0, 0)),
            pl.BlockSpec((2, OC), lambda i: (0, 0)),
        ],
        out_specs=pl.BlockSpec((TN, 3), lambda i: (i, 0)),
        out_shape=jax.ShapeDtypeStruct((R, 3), jnp.float32),
    )(g, c, fl_t, w2row, b2, acc)


def kernel(pc1, feat1, flow, W1, b1, gamma, beta, W2, b2):
    t_tab, c_tab = _k1_wrap(pc1, feat1, flow, W1, b1)
    g = _knn_gather_sc(pc1, t_tab)
    fl_t = jnp.transpose(flow, (0, 2, 1)).reshape(R, 3)
    w2row = jnp.stack([W2[0], gamma, beta], axis=0)
    out = _k3(g, c_tab, fl_t, w2row, b2.reshape(1, 1))
    return out.reshape(B, N, 3)


# speedup vs baseline: 1.9809x; 1.9809x over previous
import functools

import jax
import jax.numpy as jnp
from jax import lax
from jax.experimental import pallas as pl
from jax.experimental.pallas import tpu as pltpu
from jax.experimental.pallas import tpu_sc as plsc

B, N, D, NK = 4, 4096, 64, 32
OC = 32
TW = 48
R = B * N
M = B * N * NK
NC, NS, L = 2, 16, 16
NW = NC * NS
ROWS_PER_W = R // NW
FMAX = 3.4e38


def _k1_body(xf_ref, fl_ref, wa_ref, wc_ref, b1_ref, t_ref, c_ref):
    xf = xf_ref[0]
    a = lax.dot_general(xf, wa_ref[...], (((0,), (0,)), ((), ())),
                        preferred_element_type=jnp.float32)
    c = lax.dot_general(xf, wc_ref[...], (((0,), (0,)), ((), ())),
                        preferred_element_type=jnp.float32) + b1_ref[...]
    t_ref[0, :, 0:OC] = a
    t_ref[0, :, OC:OC + 3] = fl_ref[0].T
    t_ref[0, :, OC + 3:TW] = jnp.zeros((a.shape[0], TW - OC - 3), jnp.float32)
    c_ref[0] = c


def _k1(xf, flow, wa, wc, b1):
    TN = 1024
    grid = (B, N // TN)
    return pl.pallas_call(
        _k1_body,
        grid=grid,
        in_specs=[
            pl.BlockSpec((1, 3 + D, TN), lambda b, i: (b, 0, i)),
            pl.BlockSpec((1, 3, TN), lambda b, i: (b, 0, i)),
            pl.BlockSpec((3 + D, OC), lambda b, i: (0, 0)),
            pl.BlockSpec((3 + D, OC), lambda b, i: (0, 0)),
            pl.BlockSpec((1, OC), lambda b, i: (0, 0)),
        ],
        out_specs=[
            pl.BlockSpec((1, TN, TW), lambda b, i: (b, i, 0)),
            pl.BlockSpec((1, TN, OC), lambda b, i: (b, i, 0)),
        ],
        out_shape=[
            jax.ShapeDtypeStruct((B, N, TW), jnp.float32),
            jax.ShapeDtypeStruct((B, N, OC), jnp.float32),
        ],
    )(xf, flow, wa, wc, b1[None])


def _k1_wrap(pc1, feat1, flow, W1, b1):
    w1x = W1[:, 0:3]
    w1f = W1[:, 3:3 + D]
    w1g = W1[:, 3 + D:]
    wa = jnp.concatenate([w1x, w1f], axis=1).T
    wc = jnp.concatenate([-w1x, w1g], axis=1).T
    xf = jnp.concatenate([pc1, feat1], axis=1)
    t, c = _k1(xf, flow, wa, wc, b1)
    return t.reshape(R, TW), c.reshape(R, OC)


def _bf16_round(x):
    u = plsc.bitcast(x, jnp.uint32)
    r = u + jnp.uint32(0x7FFF) + ((u >> jnp.uint32(16)) & jnp.uint32(1))
    return plsc.bitcast(r & jnp.uint32(0xFFFF0000), jnp.float32)


def _k2_body(pc1_hbm, t_hbm, out_hbm, px, py, pz,
             pxb, pyb, pzb, s2, idxbuf, rowbuf, sem):
    wid = lax.axis_index("s") * NC + lax.axis_index("c")
    wpb = NW // B
    b = wid // wpb
    pltpu.sync_copy(pc1_hbm.at[pl.ds(b * 3 * N, N)], px)
    pltpu.sync_copy(pc1_hbm.at[pl.ds(b * 3 * N + N, N)], py)
    pltpu.sync_copy(pc1_hbm.at[pl.ds(b * 3 * N + 2 * N, N)], pz)
    row0 = wid * ROWS_PER_W
    n0 = (wid % wpb) * ROWS_PER_W
    gbase = b * N

    def pre_body(ci, _):
        base = ci * L
        cx = px[pl.ds(base, L)]
        cy = py[pl.ds(base, L)]
        cz = pz[pl.ds(base, L)]
        pxb[pl.ds(base, L)] = _bf16_round(cx)
        pyb[pl.ds(base, L)] = _bf16_round(cy)
        pzb[pl.ds(base, L)] = _bf16_round(cz)
        s2[pl.ds(base, L)] = cx * cx + cy * cy + cz * cz
        return 0

    lax.fori_loop(0, N // L, pre_body, 0, unroll=False)

    def row_body(r, _):
        n = n0 + r
        nsplat = jnp.zeros((L,), jnp.int32) + n
        pxn = plsc.load_gather(pxb, [nsplat])
        pyn = plsc.load_gather(pyb, [nsplat])
        pzn = plsc.load_gather(pzb, [nsplat])
        sn2 = plsc.load_gather(s2, [nsplat])

        def chunk_body(ci, carry):
            base = ci * L
            cx = pxb[pl.ds(base, L)]
            cy = pyb[pl.ds(base, L)]
            cz = pzb[pl.ds(base, L)]
            dot = cx * pxn + cy * pyn + cz * pzn
            d = (jnp.float32(-2.0) * dot + sn2) + s2[pl.ds(base, L)]
            r0v, r0i, r1v, r1i = carry

            def merge(c):
                r0v, r0i, r1v, r1i = c
                rcv, rci = plsc.sort_key_val(
                    d, lax.iota(jnp.int32, L) + base, descending=True)
                m = r1v <= rcv
                lv = jnp.where(m, r1v, rcv)
                li = jnp.where(m, r1i, rci)
                rlv, rli = plsc.sort_key_val(lv, li, descending=True)
                m2 = r0v <= rlv
                av = jnp.where(m2, r0v, rlv)
                ai = jnp.where(m2, r0i, rli)
                bv = jnp.where(m2, rlv, r0v)
                bi = jnp.where(m2, rli, r0i)
                nr0 = plsc.sort_key_val(av, ai)
                nr1 = plsc.sort_key_val(bv, bi)
                return nr0[0], nr0[1], nr1[0], nr1[1]

            return merge(carry)

        init = (jnp.full((L,), FMAX, jnp.float32), jnp.zeros((L,), jnp.int32),
                jnp.full((L,), FMAX, jnp.float32), jnp.zeros((L,), jnp.int32))
        r0v, r0i, r1v, r1i = lax.fori_loop(0, N // L, chunk_body, init,
                                           unroll=4)
        c0 = pltpu.async_copy(t_hbm.at[r0i + gbase],
                              rowbuf.at[pl.ds(0, L)], sem)
        c1 = pltpu.async_copy(t_hbm.at[r1i + gbase],
                              rowbuf.at[pl.ds(L, L)], sem)
        c0.wait()
        c1.wait()
        pltpu.sync_copy(rowbuf, out_hbm.at[row0 + r])
        return 0

    lax.fori_loop(0, ROWS_PER_W, row_body, 0, unroll=False)


def _knn_gather_sc(pc1, t_tab):
    mesh = plsc.VectorSubcoreMesh(core_axis_name="c", subcore_axis_name="s")
    f = functools.partial(
        pl.kernel,
        out_type=jax.ShapeDtypeStruct((R, NK, TW), jnp.float32),
        mesh=mesh,
        scratch_types=[
            pltpu.VMEM((N,), jnp.float32),
            pltpu.VMEM((N,), jnp.float32),
            pltpu.VMEM((N,), jnp.float32),
            pltpu.VMEM((N,), jnp.float32),
            pltpu.VMEM((N,), jnp.float32),
            pltpu.VMEM((N,), jnp.float32),
            pltpu.VMEM((N,), jnp.float32),
            pltpu.VMEM((NK,), jnp.int32),
            pltpu.VMEM((NK, TW), jnp.float32),
            pltpu.SemaphoreType.DMA,
        ],
        compiler_params=pltpu.CompilerParams(needs_layout_passes=False,
                                             use_tc_tiling_on_sc=False),
    )(_k2_body)
    return f(pc1.reshape(B * 3 * N), t_tab)


def _k3a_body(g_ref, c_ref, acc_ref):
    step = pl.program_id(0)

    @pl.when(step == 0)
    def _():
        acc_ref[...] = jnp.zeros_like(acc_ref)

    h = g_ref[:, :, 0:OC] + c_ref[...][:, None, :]
    acc_ref[0:1, :] += jnp.sum(h, axis=(0, 1)).reshape(1, OC)
    acc_ref[1:2, :] += jnp.sum(h * h, axis=(0, 1)).reshape(1, OC)


def _k3b_body(g_ref, c_ref, fl_ref, w2_ref, gb_ref, acc_ref, o_ref):
    h = g_ref[:, :, 0:OC] + c_ref[...][:, None, :]
    mean = acc_ref[0:1, :].reshape(1, 1, OC) * (1.0 / M)
    ex2 = acc_ref[1:2, :].reshape(1, 1, OC) * (1.0 / M)
    var = ex2 - mean * mean
    gamma = w2_ref[1:2, :].reshape(1, 1, OC)
    beta = w2_ref[2:3, :].reshape(1, 1, OC)
    w2 = w2_ref[0:1, :].reshape(1, 1, OC)
    scale = gamma * lax.rsqrt(var + 1e-5)
    hn = (h - mean) * scale + beta
    hn = jnp.where(hn >= 0, hn, 0.1 * hn)
    s = jnp.sum(hn * w2, axis=2) + gb_ref[0, 0]
    s = s - jnp.max(s, axis=1, keepdims=True)
    e = jnp.exp(s)
    w = e / jnp.sum(e, axis=1, keepdims=True)
    gfl = g_ref[:, :, OC:OC + 3]
    o_ref[...] = fl_ref[...] + jnp.sum(w[:, :, None] * gfl, axis=1)


def _k3(g, c, fl_t, w2row, b2):
    TN = 512
    nb = R // TN
    acc = pl.pallas_call(
        _k3a_body,
        grid=(nb,),
        in_specs=[
            pl.BlockSpec((TN, NK, TW), lambda i: (i, 0, 0)),
            pl.BlockSpec((TN, OC), lambda i: (i, 0)),
        ],
        out_specs=pl.BlockSpec((2, OC), lambda i: (0, 0)),
        out_shape=jax.ShapeDtypeStruct((2, OC), jnp.float32),
    )(g, c)
    return pl.pallas_call(
        _k3b_body,
        grid=(nb,),
        in_specs=[
            pl.BlockSpec((TN, NK, TW), lambda i: (i, 0, 0)),
            pl.BlockSpec((TN, OC), lambda i: (i, 0)),
            pl.BlockSpec((TN, 3), lambda i: (i, 0)),
            pl.BlockSpec((3, OC), lambda i: (0, 0)),
            pl.BlockSpec((1, 1), lambda i: (0, 0)),
            pl.BlockSpec((2, OC), lambda i: (0, 0)),
        ],
        out_specs=pl.BlockSpec((TN, 3), lambda i: (i, 0)),
        out_shape=jax.ShapeDtypeStruct((R, 3), jnp.float32),
    )(g, c, fl_t, w2row, b2, acc)


def kernel(pc1, feat1, flow, W1, b1, gamma, beta, W2, b2):
    t_tab, c_tab = _k1_wrap(pc1, feat1, flow, W1, b1)
    g = _knn_gather_sc(pc1, t_tab)
    fl_t = jnp.transpose(flow, (0, 2, 1)).reshape(R, 3)
    w2row = jnp.stack([W2[0], gamma, beta], axis=0)
    out = _k3(g, c_tab, fl_t, w2row, b2.reshape(1, 1))
    return out.reshape(B, N, 3)


# speedup vs baseline: 2.1378x; 1.0792x over previous
import functools

import jax
import jax.numpy as jnp
from jax import lax
from jax.experimental import pallas as pl
from jax.experimental.pallas import tpu as pltpu
from jax.experimental.pallas import tpu_sc as plsc

B, N, D, NK = 4, 4096, 64, 32
OC = 32
TW = 48
R = B * N
M = B * N * NK
NC, NS, L = 2, 16, 16
NW = NC * NS
ROWS_PER_W = R // NW
GROUP = 4
FMAX = 3.4e38


def _k1_body(xf_ref, fl_ref, wa_ref, wc_ref, b1_ref, t_ref, c_ref):
    xf = xf_ref[0]
    a = lax.dot_general(xf, wa_ref[...], (((0,), (0,)), ((), ())),
                        preferred_element_type=jnp.float32)
    c = lax.dot_general(xf, wc_ref[...], (((0,), (0,)), ((), ())),
                        preferred_element_type=jnp.float32) + b1_ref[...]
    t_ref[0, :, 0:OC] = a
    t_ref[0, :, OC:OC + 3] = fl_ref[0].T
    t_ref[0, :, OC + 3:TW] = jnp.zeros((a.shape[0], TW - OC - 3), jnp.float32)
    c_ref[0] = c


def _k1(xf, flow, wa, wc, b1):
    TN = 1024
    grid = (B, N // TN)
    return pl.pallas_call(
        _k1_body,
        grid=grid,
        in_specs=[
            pl.BlockSpec((1, 3 + D, TN), lambda b, i: (b, 0, i)),
            pl.BlockSpec((1, 3, TN), lambda b, i: (b, 0, i)),
            pl.BlockSpec((3 + D, OC), lambda b, i: (0, 0)),
            pl.BlockSpec((3 + D, OC), lambda b, i: (0, 0)),
            pl.BlockSpec((1, OC), lambda b, i: (0, 0)),
        ],
        out_specs=[
            pl.BlockSpec((1, TN, TW), lambda b, i: (b, i, 0)),
            pl.BlockSpec((1, TN, OC), lambda b, i: (b, i, 0)),
        ],
        out_shape=[
            jax.ShapeDtypeStruct((B, N, TW), jnp.float32),
            jax.ShapeDtypeStruct((B, N, OC), jnp.float32),
        ],
    )(xf, flow, wa, wc, b1[None])


def _k1_wrap(pc1, feat1, flow, W1, b1):
    w1x = W1[:, 0:3]
    w1f = W1[:, 3:3 + D]
    w1g = W1[:, 3 + D:]
    wa = jnp.concatenate([w1x, w1f], axis=1).T
    wc = jnp.concatenate([-w1x, w1g], axis=1).T
    xf = jnp.concatenate([pc1, feat1], axis=1)
    t, c = _k1(xf, flow, wa, wc, b1)
    return t.reshape(R, TW), c.reshape(R, OC)


def _bf16_round(x):
    u = plsc.bitcast(x, jnp.uint32)
    r = u + jnp.uint32(0x7FFF) + ((u >> jnp.uint32(16)) & jnp.uint32(1))
    return plsc.bitcast(r & jnp.uint32(0xFFFF0000), jnp.float32)


def _k2_body(pc1_hbm, t_hbm, out_hbm, px, py, pz,
             pxb, pyb, pzb, s2, idxbuf, rowbuf, sem):
    wid = lax.axis_index("s") * NC + lax.axis_index("c")
    wpb = NW // B
    b = wid // wpb
    pltpu.sync_copy(pc1_hbm.at[pl.ds(b * 3 * N, N)], px)
    pltpu.sync_copy(pc1_hbm.at[pl.ds(b * 3 * N + N, N)], py)
    pltpu.sync_copy(pc1_hbm.at[pl.ds(b * 3 * N + 2 * N, N)], pz)
    row0 = wid * ROWS_PER_W
    n0 = (wid % wpb) * ROWS_PER_W
    gbase = b * N

    def pre_body(ci, _):
        base = ci * L
        cx = px[pl.ds(base, L)]
        cy = py[pl.ds(base, L)]
        cz = pz[pl.ds(base, L)]
        pxb[pl.ds(base, L)] = _bf16_round(cx)
        pyb[pl.ds(base, L)] = _bf16_round(cy)
        pzb[pl.ds(base, L)] = _bf16_round(cz)
        s2[pl.ds(base, L)] = cx * cx + cy * cy + cz * cz
        return 0

    lax.fori_loop(0, N // L, pre_body, 0, unroll=False)

    def topk_row(n):
        nsplat = jnp.zeros((L,), jnp.int32) + n
        pxn = plsc.load_gather(pxb, [nsplat])
        pyn = plsc.load_gather(pyb, [nsplat])
        pzn = plsc.load_gather(pzb, [nsplat])
        sn2 = plsc.load_gather(s2, [nsplat])

        def chunk_body(ci, carry):
            base = ci * L
            cx = pxb[pl.ds(base, L)]
            cy = pyb[pl.ds(base, L)]
            cz = pzb[pl.ds(base, L)]
            dot = cx * pxn + cy * pyn + cz * pzn
            d = (jnp.float32(-2.0) * dot + sn2) + s2[pl.ds(base, L)]
            r0v, r0i, r1v, r1i = carry

            def merge(c):
                r0v, r0i, r1v, r1i = c
                rcv, rci = plsc.sort_key_val(
                    d, lax.iota(jnp.int32, L) + base, descending=True)
                m = r1v <= rcv
                lv = jnp.where(m, r1v, rcv)
                li = jnp.where(m, r1i, rci)
                rlv, rli = plsc.sort_key_val(lv, li, descending=True)
                m2 = r0v <= rlv
                av = jnp.where(m2, r0v, rlv)
                ai = jnp.where(m2, r0i, rli)
                bv = jnp.where(m2, rlv, r0v)
                bi = jnp.where(m2, rli, r0i)
                nr0 = plsc.sort_key_val(av, ai)
                nr1 = plsc.sort_key_val(bv, bi)
                return nr0[0], nr0[1], nr1[0], nr1[1]

            return merge(carry)

        init = (jnp.full((L,), FMAX, jnp.float32), jnp.zeros((L,), jnp.int32),
                jnp.full((L,), FMAX, jnp.float32), jnp.zeros((L,), jnp.int32))
        r0v, r0i, r1v, r1i = lax.fori_loop(0, N // L, chunk_body, init,
                                           unroll=False)
        return r0i, r1i

    def group_body(g, _):
        for j in range(GROUP):
            r = g * GROUP + j
            r0i, r1i = topk_row(n0 + r)
            idxbuf[pl.ds(j * NK, L)] = r0i + gbase
            idxbuf[pl.ds(j * NK + L, L)] = r1i + gbase
        pltpu.async_copy(t_hbm.at[idxbuf], rowbuf, sem).wait()
        pltpu.sync_copy(
            rowbuf, out_hbm.at[pl.ds((row0 + g * GROUP) * NK, GROUP * NK)])
        return 0

    lax.fori_loop(0, ROWS_PER_W // GROUP, group_body, 0, unroll=False)


def _knn_gather_sc(pc1, t_tab):
    mesh = plsc.VectorSubcoreMesh(core_axis_name="c", subcore_axis_name="s")
    f = functools.partial(
        pl.kernel,
        out_type=jax.ShapeDtypeStruct((R * NK, TW), jnp.float32),
        mesh=mesh,
        scratch_types=[
            pltpu.VMEM((N,), jnp.float32),
            pltpu.VMEM((N,), jnp.float32),
            pltpu.VMEM((N,), jnp.float32),
            pltpu.VMEM((N,), jnp.float32),
            pltpu.VMEM((N,), jnp.float32),
            pltpu.VMEM((N,), jnp.float32),
            pltpu.VMEM((N,), jnp.float32),
            pltpu.VMEM((GROUP * NK,), jnp.int32),
            pltpu.VMEM((GROUP * NK, TW), jnp.float32),
            pltpu.SemaphoreType.DMA,
        ],
        compiler_params=pltpu.CompilerParams(needs_layout_passes=False,
                                             use_tc_tiling_on_sc=False),
    )(_k2_body)
    return f(pc1.reshape(B * 3 * N), t_tab).reshape(R, NK, TW)


def _k3a_body(g_ref, c_ref, acc_ref):
    step = pl.program_id(0)

    @pl.when(step == 0)
    def _():
        acc_ref[...] = jnp.zeros_like(acc_ref)

    h = g_ref[:, :, 0:OC] + c_ref[...][:, None, :]
    acc_ref[0:1, :] += jnp.sum(h, axis=(0, 1)).reshape(1, OC)
    acc_ref[1:2, :] += jnp.sum(h * h, axis=(0, 1)).reshape(1, OC)


def _k3b_body(g_ref, c_ref, fl_ref, w2_ref, gb_ref, acc_ref, o_ref):
    h = g_ref[:, :, 0:OC] + c_ref[...][:, None, :]
    mean = acc_ref[0:1, :].reshape(1, 1, OC) * (1.0 / M)
    ex2 = acc_ref[1:2, :].reshape(1, 1, OC) * (1.0 / M)
    var = ex2 - mean * mean
    gamma = w2_ref[1:2, :].reshape(1, 1, OC)
    beta = w2_ref[2:3, :].reshape(1, 1, OC)
    w2 = w2_ref[0:1, :].reshape(1, 1, OC)
    scale = gamma * lax.rsqrt(var + 1e-5)
    hn = (h - mean) * scale + beta
    hn = jnp.where(hn >= 0, hn, 0.1 * hn)
    s = jnp.sum(hn * w2, axis=2) + gb_ref[0, 0]
    s = s - jnp.max(s, axis=1, keepdims=True)
    e = jnp.exp(s)
    w = e / jnp.sum(e, axis=1, keepdims=True)
    gfl = g_ref[:, :, OC:OC + 3]
    o_ref[...] = fl_ref[...] + jnp.sum(w[:, :, None] * gfl, axis=1)


def _k3(g, c, fl_t, w2row, b2):
    TN = 512
    nb = R // TN
    acc = pl.pallas_call(
        _k3a_body,
        grid=(nb,),
        in_specs=[
            pl.BlockSpec((TN, NK, TW), lambda i: (i, 0, 0)),
            pl.BlockSpec((TN, OC), lambda i: (i, 0)),
        ],
        out_specs=pl.BlockSpec((2, OC), lambda i: (0, 0)),
        out_shape=jax.ShapeDtypeStruct((2, OC), jnp.float32),
    )(g, c)
    return pl.pallas_call(
        _k3b_body,
        grid=(nb,),
        in_specs=[
            pl.BlockSpec((TN, NK, TW), lambda i: (i, 0, 0)),
            pl.BlockSpec((TN, OC), lambda i: (i, 0)),
            pl.BlockSpec((TN, 3), lambda i: (i, 0)),
            pl.BlockSpec((3, OC), lambda i: (0, 0)),
            pl.BlockSpec((1, 1), lambda i: (0, 0)),
            pl.BlockSpec((2, OC), lambda i: (0, 0)),
        ],
        out_specs=pl.BlockSpec((TN, 3), lambda i: (i, 0)),
        out_shape=jax.ShapeDtypeStruct((R, 3), jnp.float32),
    )(g, c, fl_t, w2row, b2, acc)


def kernel(pc1, feat1, flow, W1, b1, gamma, beta, W2, b2):
    t_tab, c_tab = _k1_wrap(pc1, feat1, flow, W1, b1)
    g = _knn_gather_sc(pc1, t_tab)
    fl_t = jnp.transpose(flow, (0, 2, 1)).reshape(R, 3)
    w2row = jnp.stack([W2[0], gamma, beta], axis=0)
    out = _k3(g, c_tab, fl_t, w2row, b2.reshape(1, 1))
    return out.reshape(B, N, 3)
